# baseline (speedup 1.0000x reference)
import functools

import jax
import jax.numpy as jnp
from jax import lax
from jax.experimental import pallas as pl
from jax.experimental.pallas import tpu as pltpu
from jax.experimental.pallas import tpu_sc as plsc

EMBED = 64
BATCH = 16384
NC = 2
NS = 16
NW = NC * NS
B_PER_W = BATCH // NW
IDX_CHUNK = 128
N_CHUNKS = B_PER_W // IDX_CHUNK


def _gather_body(uid_hbm, iid_hbm, utab_hbm, itab_hbm, uvec_hbm, ivec_hbm,
                 uidx_v, iidx_v, urows_v, irows_v, sem):
    wid = lax.axis_index("s") * NC + lax.axis_index("c")
    base = wid * B_PER_W
    row_base = wid * N_CHUNKS
    pltpu.sync_copy(uid_hbm.at[pl.ds(row_base, N_CHUNKS)], uidx_v)
    pltpu.sync_copy(iid_hbm.at[pl.ds(row_base, N_CHUNKS)], iidx_v)
    copies = []
    for j in range(N_CHUNKS):
        copies.append(pltpu.async_copy(
            utab_hbm.at[uidx_v.at[j]],
            urows_v.at[pl.ds(j * IDX_CHUNK, IDX_CHUNK)], sem))
        copies.append(pltpu.async_copy(
            itab_hbm.at[iidx_v.at[j]],
            irows_v.at[pl.ds(j * IDX_CHUNK, IDX_CHUNK)], sem))
    for c in copies:
        c.wait()
    pltpu.sync_copy(urows_v, uvec_hbm.at[pl.ds(base, B_PER_W)])
    pltpu.sync_copy(irows_v, ivec_hbm.at[pl.ds(base, B_PER_W)])


@functools.cache
def _make_gather():
    return pl.kernel(
        _gather_body,
        mesh=plsc.VectorSubcoreMesh(core_axis_name="c", subcore_axis_name="s"),
        out_type=[
            jax.ShapeDtypeStruct((BATCH, EMBED), jnp.float32),
            jax.ShapeDtypeStruct((BATCH, EMBED), jnp.float32),
        ],
        scratch_types=[
            pltpu.VMEM((N_CHUNKS, IDX_CHUNK), jnp.int32),
            pltpu.VMEM((N_CHUNKS, IDX_CHUNK), jnp.int32),
            pltpu.VMEM((B_PER_W, EMBED), jnp.float32),
            pltpu.VMEM((B_PER_W, EMBED), jnp.float32),
            pltpu.SemaphoreType.DMA,
        ],
        compiler_params=pltpu.CompilerParams(use_tc_tiling_on_sc=False),
    )


def _mlp_body(uvec_ref, ivec_ref, w1u_ref, w1i_ref, b1_ref, w2_ref, b2_ref,
              w3_ref, b3_ref, out_ref):
    h = jnp.dot(uvec_ref[...], w1u_ref[...], preferred_element_type=jnp.float32)
    h = h + jnp.dot(ivec_ref[...], w1i_ref[...],
                    preferred_element_type=jnp.float32)
    h = jnp.maximum(h + b1_ref[...], 0.0)
    h = jnp.maximum(
        jnp.dot(h, w2_ref[...], preferred_element_type=jnp.float32)
        + b2_ref[...], 0.0)
    out_ref[...] = jnp.sum(h * w3_ref[...], axis=1) + b3_ref[0]


MLP_BLK = 2048


def _mlp(uvec, ivec, w1u, w1i, b1, w2t, b2, w3, b3):
    grid = (BATCH // MLP_BLK,)
    return pl.pallas_call(
        _mlp_body,
        grid=grid,
        in_specs=[
            pl.BlockSpec((MLP_BLK, EMBED), lambda i: (i, 0)),
            pl.BlockSpec((MLP_BLK, EMBED), lambda i: (i, 0)),
            pl.BlockSpec((EMBED, 128), lambda i: (0, 0)),
            pl.BlockSpec((EMBED, 128), lambda i: (0, 0)),
            pl.BlockSpec((1, 128), lambda i: (0, 0)),
            pl.BlockSpec((128, EMBED), lambda i: (0, 0)),
            pl.BlockSpec((1, EMBED), lambda i: (0, 0)),
            pl.BlockSpec((1, EMBED), lambda i: (0, 0)),
            pl.BlockSpec(memory_space=pltpu.SMEM),
        ],
        out_specs=pl.BlockSpec((MLP_BLK,), lambda i: (i,)),
        out_shape=jax.ShapeDtypeStruct((BATCH,), jnp.float32),
    )(uvec, ivec, w1u, w1i, b1, w2t, b2, w3, b3)


def kernel(user_ids, item_ids, user_table, item_table, W1, b1, W2, b2, W3, b3):
    uid2 = user_ids.astype(jnp.int32).reshape(BATCH // IDX_CHUNK, IDX_CHUNK)
    iid2 = item_ids.astype(jnp.int32).reshape(BATCH // IDX_CHUNK, IDX_CHUNK)
    uvec, ivec = _make_gather()(uid2, iid2, user_table, item_table)
    w1u = W1[:, :EMBED].T
    w1i = W1[:, EMBED:].T
    return _mlp(uvec, ivec, w1u, w1i, b1.reshape(1, 128), W2.T,
                b2.reshape(1, EMBED), W3, b3)


# speedup vs baseline: 1.9603x; 1.9603x over previous
import functools

import jax
import jax.numpy as jnp
from jax import lax
from jax.experimental import pallas as pl
from jax.experimental.pallas import tpu as pltpu
from jax.experimental.pallas import tpu_sc as plsc

EMBED = 64
BATCH = 16384
NROWS = 1000000
NC = 2
NS = 16
NW = NC * NS
B_PER_W = BATCH // NW
IDX_CHUNK = 128
N_CHUNKS = B_PER_W // IDX_CHUNK

PACK_L = 2048


def _pack_body(u_ref, i_ref, out_ref):
    x = jnp.concatenate([u_ref[...], i_ref[...]], axis=0)
    out_ref[...] = jnp.swapaxes(x, 0, 1)


def _pack(u_t, i_t):
    grid = (pl.cdiv(NROWS, PACK_L),)
    return pl.pallas_call(
        _pack_body,
        grid=grid,
        in_specs=[
            pl.BlockSpec((EMBED, PACK_L), lambda k: (0, k)),
            pl.BlockSpec((EMBED, PACK_L), lambda k: (0, k)),
        ],
        out_specs=pl.BlockSpec((PACK_L, 2 * EMBED), lambda k: (k, 0)),
        out_shape=jax.ShapeDtypeStruct((NROWS, 2 * EMBED), jnp.float32),
    )(u_t, i_t)


def _gather_body(uid_hbm, iid_hbm, packed_hbm, xu_hbm, xi_hbm,
                 uidx_v, iidx_v, rows_v, sem):
    wid = lax.axis_index("s") * NC + lax.axis_index("c")
    base = wid * B_PER_W
    row_base = wid * N_CHUNKS
    pltpu.sync_copy(uid_hbm.at[pl.ds(row_base, N_CHUNKS)], uidx_v)
    pltpu.sync_copy(iid_hbm.at[pl.ds(row_base, N_CHUNKS)], iidx_v)
    for idx_v, dst in ((uidx_v, xu_hbm), (iidx_v, xi_hbm)):
        copies = []
        for j in range(N_CHUNKS):
            copies.append(pltpu.async_copy(
                packed_hbm.at[idx_v.at[j]],
                rows_v.at[pl.ds(j * IDX_CHUNK, IDX_CHUNK)], sem))
        for c in copies:
            c.wait()
        pltpu.sync_copy(rows_v, dst.at[pl.ds(base, B_PER_W)])


@functools.cache
def _make_gather():
    return pl.kernel(
        _gather_body,
        mesh=plsc.VectorSubcoreMesh(core_axis_name="c", subcore_axis_name="s"),
        out_type=[
            jax.ShapeDtypeStruct((BATCH, 2 * EMBED), jnp.float32),
            jax.ShapeDtypeStruct((BATCH, 2 * EMBED), jnp.float32),
        ],
        scratch_types=[
            pltpu.VMEM((N_CHUNKS, IDX_CHUNK), jnp.int32),
            pltpu.VMEM((N_CHUNKS, IDX_CHUNK), jnp.int32),
            pltpu.VMEM((B_PER_W, 2 * EMBED), jnp.float32),
            pltpu.SemaphoreType.DMA,
        ],
    )


def _mlp_body(xu_ref, xi_ref, wa_ref, wb_ref, b1_ref, w2_ref, b2_ref,
              w3_ref, b3_ref, out_ref):
    h = jnp.dot(xu_ref[...], wa_ref[...], preferred_element_type=jnp.float32)
    h = h + jnp.dot(xi_ref[...], wb_ref[...],
                    preferred_element_type=jnp.float32)
    h = jnp.maximum(h + b1_ref[...], 0.0)
    h = jnp.maximum(
        jnp.dot(h, w2_ref[...], preferred_element_type=jnp.float32)
        + b2_ref[...], 0.0)
    out_ref[...] = jnp.sum(h * w3_ref[...], axis=1) + b3_ref[0]


MLP_BLK = 2048


def _mlp(xu, xi, wa, wb, b1, w2t, b2, w3, b3):
    grid = (BATCH // MLP_BLK,)
    return pl.pallas_call(
        _mlp_body,
        grid=grid,
        in_specs=[
            pl.BlockSpec((MLP_BLK, 2 * EMBED), lambda i: (i, 0)),
            pl.BlockSpec((MLP_BLK, 2 * EMBED), lambda i: (i, 0)),
            pl.BlockSpec((2 * EMBED, 128), lambda i: (0, 0)),
            pl.BlockSpec((2 * EMBED, 128), lambda i: (0, 0)),
            pl.BlockSpec((1, 128), lambda i: (0, 0)),
            pl.BlockSpec((128, EMBED), lambda i: (0, 0)),
            pl.BlockSpec((1, EMBED), lambda i: (0, 0)),
            pl.BlockSpec((1, EMBED), lambda i: (0, 0)),
            pl.BlockSpec(memory_space=pltpu.SMEM),
        ],
        out_specs=pl.BlockSpec((MLP_BLK,), lambda i: (i,)),
        out_shape=jax.ShapeDtypeStruct((BATCH,), jnp.float32),
    )(xu, xi, wa, wb, b1, w2t, b2, w3, b3)


def kernel(user_ids, item_ids, user_table, item_table, W1, b1, W2, b2, W3, b3):
    packed = _pack(user_table.T, item_table.T)
    uid2 = user_ids.astype(jnp.int32).reshape(BATCH // IDX_CHUNK, IDX_CHUNK)
    iid2 = item_ids.astype(jnp.int32).reshape(BATCH // IDX_CHUNK, IDX_CHUNK)
    xu, xi = _make_gather()(uid2, iid2, packed)
    zeros = jnp.zeros((EMBED, 128), jnp.float32)
    wa = jnp.concatenate([W1[:, :EMBED].T, zeros], axis=0)
    wb = jnp.concatenate([zeros, W1[:, EMBED:].T], axis=0)
    return _mlp(xu, xi, wa, wb, b1.reshape(1, 128), W2.T,
                b2.reshape(1, EMBED), W3, b3)
